# baseline (speedup 1.0000x reference)
import functools

import jax
import jax.numpy as jnp
from jax import lax
from jax.experimental import pallas as pl
from jax.experimental.pallas import tpu as pltpu, tpu_sc as plsc

N = 10000
E = 320000
D = 128
G = 64
NP = 10240
NS = 16
NC = 2
SA = NP // NS
SY = NP // (NS * NC)
SB = NP // NS
EPT_B = E // (NS * NC)
CH = 128
CHH = 128

_mesh = plsc.VectorSubcoreMesh(core_axis_name="c", subcore_axis_name="s")
_sc_params = pltpu.CompilerParams(use_tc_tiling_on_sc=False)


def _rsqrt16(v):
    i = lax.bitcast_convert_type(v, jnp.int32)
    i = jnp.int32(0x5F3759DF) - lax.shift_right_logical(i, 1)
    z = lax.bitcast_convert_type(i, jnp.float32)
    for _ in range(3):
        z = z * (1.5 - 0.5 * v * z * z)
    return z


@functools.partial(
    pl.kernel,
    out_type=jax.ShapeDtypeStruct((NC * NP, 16), jnp.float32),
    mesh=_mesh,
    compiler_params=_sc_params,
    scratch_types=[
        pltpu.MemorySpace.VMEM_SHARED((NP, 16), jnp.float32),
        pltpu.VMEM((CHH,), jnp.int32),
        pltpu.VMEM((CHH,), jnp.int32),
        pltpu.VMEM((16,), jnp.int32),
        pltpu.VMEM((CHH, 16), jnp.float32),
        pltpu.VMEM((16, 16), jnp.float32),
        pltpu.VMEM((SA, 16), jnp.float32),
        pltpu.SemaphoreType.DMA,
        pltpu.SemaphoreType.DMA,
    ],
)
def _sc_hist(dst_hbm, deg_hbm,
             hist_sp, didx0, didx1, didx_t, ones, ones_t, hrows, s0, s1):
    cid = lax.axis_index("c")
    sid = lax.axis_index("s")
    zeros16 = jnp.zeros((16,), jnp.float32)
    ones16 = jnp.ones((16,), jnp.float32)
    NCH = EPT_B // CHH

    @pl.loop(0, SA)
    def _(r):
        hrows[r, :] = zeros16

    @pl.loop(0, CHH)
    def _(r):
        ones[r, :] = ones16

    @pl.loop(0, 16)
    def _(r):
        ones_t[r, :] = ones16

    pltpu.sync_copy(hrows, hist_sp.at[pl.ds(sid * SA, SA), :])
    plsc.subcore_barrier()

    ebase = cid * (E // NC) + sid * EPT_B
    pltpu.sync_copy(dst_hbm.at[pl.ds(ebase, CHH)], didx0)

    @pl.loop(0, NCH // 2)
    def _(c):
        pltpu.async_copy(ones, hist_sp.at[didx0], s0, add=True)
        pltpu.sync_copy(dst_hbm.at[pl.ds(ebase + (2 * c + 1) * CHH, CHH)],
                        didx1)
        pltpu.async_copy(ones, hist_sp.at[didx1], s1, add=True)
        pltpu.make_async_copy(ones, hist_sp.at[didx0], s0).wait()

        @pl.when(c < NCH // 2 - 1)
        def _():
            pltpu.sync_copy(dst_hbm.at[pl.ds(ebase + (2 * c + 2) * CHH, CHH)],
                            didx0)

        pltpu.make_async_copy(ones, hist_sp.at[didx1], s1).wait()

    n_tail = EPT_B - NCH * CHH
    pltpu.sync_copy(dst_hbm.at[pl.ds(ebase + EPT_B - n_tail, n_tail)], didx_t)
    pltpu.sync_copy(ones_t, hist_sp.at[didx_t], add=True)
    plsc.subcore_barrier()

    pltpu.sync_copy(hist_sp.at[pl.ds(sid * SA, SA), :], hrows)
    pltpu.sync_copy(hrows, deg_hbm.at[pl.ds(cid * NP + sid * SA, SA), :])


@functools.partial(
    pl.kernel,
    out_type=(
        jax.ShapeDtypeStruct((NP, D), jnp.float32),
        jax.ShapeDtypeStruct((NP, 16), jnp.float32),
    ),
    mesh=_mesh,
    compiler_params=_sc_params,
    scratch_types=[
        pltpu.VMEM((SY, 16), jnp.float32),
        pltpu.VMEM((SY, 16), jnp.float32),
        pltpu.VMEM((SY, D), jnp.float32),
        pltpu.SemaphoreType.DMA,
    ],
)
def _sc_y(x_hbm, deg_hbm, y_hbm, dis_hbm, h0, h1, xbuf, xs):
    cid = lax.axis_index("c")
    sid = lax.axis_index("s")
    wid = sid * NC + cid
    rbase = wid * SY
    last = NS * NC - 1
    SL = N - last * SY

    @pl.when(wid < last)
    def _():
        pltpu.async_copy(x_hbm.at[pl.ds(rbase, SY), :], xbuf, xs)

    @pl.when(wid == last)
    def _():
        pltpu.async_copy(x_hbm.at[pl.ds(rbase, SL), :],
                         xbuf.at[pl.ds(0, SL), :], xs)

    pltpu.sync_copy(deg_hbm.at[pl.ds(rbase, SY), :], h0)
    pltpu.sync_copy(deg_hbm.at[pl.ds(NP + rbase, SY), :], h1)

    @pl.loop(0, SY)
    def _(r):
        disv = _rsqrt16(h0[r, :] + h1[r, :] + 1.0)
        h0[r, :] = disv

    pltpu.sync_copy(h0, dis_hbm.at[pl.ds(rbase, SY), :])
    nrows = jnp.where(wid == last, SL, SY)

    @pl.when(wid < last)
    def _():
        pltpu.make_async_copy(x_hbm.at[pl.ds(rbase, SY), :], xbuf, xs).wait()

    @pl.when(wid == last)
    def _():
        pltpu.make_async_copy(x_hbm.at[pl.ds(rbase, SL), :],
                              xbuf.at[pl.ds(0, SL), :], xs).wait()

    @pl.loop(0, nrows)
    def _(r):
        disv = h0[r, :]
        for k in range(D // 16):
            xbuf[r, pl.ds(k * 16, 16)] = xbuf[r, pl.ds(k * 16, 16)] * disv

    @pl.when(wid < last)
    def _():
        pltpu.sync_copy(xbuf, y_hbm.at[pl.ds(rbase, SY), :])

    @pl.when(wid == last)
    def _():
        pltpu.sync_copy(xbuf.at[pl.ds(0, SL), :],
                        y_hbm.at[pl.ds(rbase, SL), :])


@functools.partial(
    pl.kernel,
    out_type=jax.ShapeDtypeStruct((NC * NP, D), jnp.float32),
    mesh=_mesh,
    compiler_params=_sc_params,
    scratch_types=[
        pltpu.MemorySpace.VMEM_SHARED((NP, D), jnp.float32),
        pltpu.VMEM((2, CH), jnp.int32),
        pltpu.VMEM((2, CH), jnp.int32),
        pltpu.VMEM((2, 16), jnp.int32),
        pltpu.VMEM((CH, D), jnp.float32),
        pltpu.VMEM((CH, D), jnp.float32),
        pltpu.VMEM((16, D), jnp.float32),
        pltpu.SemaphoreType.DMA,
        pltpu.SemaphoreType.DMA,
        pltpu.SemaphoreType.DMA,
        pltpu.SemaphoreType.DMA,
    ],
)
def _sc_agg(y_hbm, eidx_hbm, acc_hbm,
            acc_sp, eidx0, eidx1, eidx_t,
            rows0, rows1, rows_t, g0, g1, s0, s1):
    cid = lax.axis_index("c")
    sid = lax.axis_index("s")
    zeros16 = jnp.zeros((16,), jnp.float32)
    NCH = EPT_B // CH

    @pl.loop(0, CH)
    def _(r):
        for k in range(D // 16):
            rows0[r, pl.ds(k * 16, 16)] = zeros16

    for off in range(0, SB, CH):
        pltpu.sync_copy(rows0, acc_sp.at[pl.ds(sid * SB + off, CH), :])
    plsc.subcore_barrier()

    ebase = cid * (E // NC) + sid * EPT_B

    def load_idx(c, eidx):
        pltpu.sync_copy(eidx_hbm.at[:, pl.ds(ebase + c * CH, CH)], eidx)

    load_idx(0, eidx0)
    pltpu.async_copy(y_hbm.at[eidx0.at[0]], rows0, g0)
    load_idx(1, eidx1)
    pltpu.async_copy(y_hbm.at[eidx1.at[0]], rows1, g1)

    @pl.loop(0, NCH // 2)
    def _(c):
        pltpu.make_async_copy(y_hbm.at[eidx0.at[0]], rows0, g0).wait()
        pltpu.async_copy(rows0, acc_sp.at[eidx0.at[1]], s0, add=True)
        pltpu.make_async_copy(y_hbm.at[eidx1.at[0]], rows1, g1).wait()
        pltpu.async_copy(rows1, acc_sp.at[eidx1.at[1]], s1, add=True)

        @pl.when(c < NCH // 2 - 1)
        def _():
            pltpu.make_async_copy(rows0, acc_sp.at[eidx0.at[1]], s0).wait()
            load_idx(2 * c + 2, eidx0)
            pltpu.async_copy(y_hbm.at[eidx0.at[0]], rows0, g0)
            pltpu.make_async_copy(rows1, acc_sp.at[eidx1.at[1]], s1).wait()
            load_idx(2 * c + 3, eidx1)
            pltpu.async_copy(y_hbm.at[eidx1.at[0]], rows1, g1)

    pltpu.make_async_copy(rows0, acc_sp.at[eidx0.at[1]], s0).wait()
    pltpu.make_async_copy(rows1, acc_sp.at[eidx1.at[1]], s1).wait()

    n_tail = EPT_B - NCH * CH
    b = ebase + EPT_B - n_tail
    pltpu.sync_copy(eidx_hbm.at[:, pl.ds(b, n_tail)], eidx_t)
    pltpu.async_copy(y_hbm.at[eidx_t.at[0]], rows_t, g0).wait()
    pltpu.sync_copy(rows_t, acc_sp.at[eidx_t.at[1]], add=True)
    plsc.subcore_barrier()

    rbase = sid * SB
    for off in range(0, SB, CH):
        pltpu.sync_copy(acc_sp.at[pl.ds(rbase + off, CH), :], rows0)
        pltpu.sync_copy(rows0, acc_hbm.at[pl.ds(cid * NP + rbase + off, CH), :])


RB = 1000
NBK = N // RB

def _tc_body(y_ref, dis_ref, acc_ref, batch_ref,
             wg_ref, bg_ref, gm_ref, bt_ref, wf_ref, bf_ref, o_ref,
             sout, s1v, s2v, pool, cntv):
    i = pl.program_id(0)

    @pl.when(i == 0)
    def _():
        s1v[...] = jnp.zeros((1, D), jnp.float32)
        s2v[...] = jnp.zeros((1, D), jnp.float32)
        pool[...] = jnp.zeros((G, D), jnp.float32)
        cntv[...] = jnp.zeros((G, 1), jnp.float32)

    @pl.when(i < NBK)
    def _():
        z = dis_ref[...][:, 0:1] * (acc_ref[0] + acc_ref[1] + y_ref[...])
        ob = (jnp.dot(z, wg_ref[...], preferred_element_type=jnp.float32)
              + bg_ref[...])
        sout[pl.ds(i * RB, RB), :] = ob
        s1v[...] += jnp.sum(ob, axis=0, keepdims=True)
        s2v[...] += jnp.sum(ob * ob, axis=0, keepdims=True)

    @pl.when(i == NBK)
    def _():
        mean = s1v[...] / N
        var = s2v[...] / N - mean * mean
        a = lax.rsqrt(var + 1e-5) * gm_ref[...]
        s1v[...] = a
        s2v[...] = bt_ref[...] - mean * a

    @pl.when(i >= NBK)
    def _():
        j = i - NBK
        ob = sout[pl.ds(j * RB, RB), :]
        xb = ob * s1v[...] + s2v[...]
        xe = jnp.where(xb > 0, xb, jnp.exp(jnp.minimum(xb, 0.0)) - 1.0)
        onehot = (lax.broadcasted_iota(jnp.int32, (G, RB), 0)
                  == batch_ref[0]).astype(jnp.float32)
        pool[...] += jnp.dot(onehot, xe, preferred_element_type=jnp.float32)
        cntv[...] += jnp.sum(onehot, axis=1, keepdims=True)

    @pl.when(i == 2 * NBK - 1)
    def _():
        pooled = pool[...] / jnp.maximum(cntv[...], 1.0)
        o_ref[...] = (jnp.dot(pooled, wf_ref[...],
                              preferred_element_type=jnp.float32)
                      + bf_ref[...])


def _p1_map(i):
    return (jnp.minimum(i, NBK - 1), 0)


_tc_final = pl.pallas_call(
    _tc_body,
    grid=(2 * NBK,),
    in_specs=[
        pl.BlockSpec((RB, D), _p1_map),
        pl.BlockSpec((RB, 16), _p1_map),
        pl.BlockSpec((NC, RB, D), lambda i: (0, jnp.minimum(i, NBK - 1), 0)),
        pl.BlockSpec((1, 1, RB), lambda i: (jnp.maximum(i - NBK, 0), 0, 0)),
        pl.BlockSpec((D, D), lambda i: (0, 0)),
        pl.BlockSpec((1, D), lambda i: (0, 0)),
        pl.BlockSpec((1, D), lambda i: (0, 0)),
        pl.BlockSpec((1, D), lambda i: (0, 0)),
        pl.BlockSpec((D, D), lambda i: (0, 0)),
        pl.BlockSpec((1, D), lambda i: (0, 0)),
    ],
    out_specs=pl.BlockSpec((G, D), lambda i: (0, 0)),
    out_shape=jax.ShapeDtypeStruct((G, D), jnp.float32),
    scratch_shapes=[
        pltpu.VMEM((N, D), jnp.float32),
        pltpu.VMEM((1, D), jnp.float32),
        pltpu.VMEM((1, D), jnp.float32),
        pltpu.VMEM((G, D), jnp.float32),
        pltpu.VMEM((G, 1), jnp.float32),
    ],
)


def kernel(x, edge_index, batch, W_gcn, b_gcn, gamma, beta, W_fc, b_fc):
    dst = edge_index[1]
    degp = _sc_hist(dst)
    y, dis = _sc_y(x, degp)
    acc = _sc_agg(y, edge_index)
    return _tc_final(y, dis, acc.reshape(NC, NP, D),
                     batch.reshape(NBK, 1, RB), W_gcn, b_gcn.reshape(1, D),
                     gamma.reshape(1, D), beta.reshape(1, D),
                     W_fc, b_fc.reshape(1, D))


# speedup vs baseline: 1.0876x; 1.0876x over previous
import functools

import jax
import jax.numpy as jnp
from jax import lax
from jax.experimental import pallas as pl
from jax.experimental.pallas import tpu as pltpu, tpu_sc as plsc

N = 10000
E = 320000
D = 128
G = 64
NP = 10240
NS = 16
NC = 2
SA = NP // NS
SY = NP // (NS * NC)
SB = NP // NS
EPT_B = E // (NS * NC)
CH = 128
CHH = 128

_mesh = plsc.VectorSubcoreMesh(core_axis_name="c", subcore_axis_name="s")
_sc_params = pltpu.CompilerParams(use_tc_tiling_on_sc=False)


def _rsqrt16(v):
    i = lax.bitcast_convert_type(v, jnp.int32)
    i = jnp.int32(0x5F3759DF) - lax.shift_right_logical(i, 1)
    z = lax.bitcast_convert_type(i, jnp.float32)
    for _ in range(3):
        z = z * (1.5 - 0.5 * v * z * z)
    return z


@functools.partial(
    pl.kernel,
    out_type=jax.ShapeDtypeStruct((NC * NP, 16), jnp.float32),
    mesh=_mesh,
    compiler_params=_sc_params,
    scratch_types=[
        pltpu.MemorySpace.VMEM_SHARED((NP, 16), jnp.float32),
        pltpu.VMEM((CHH,), jnp.int32),
        pltpu.VMEM((CHH,), jnp.int32),
        pltpu.VMEM((16,), jnp.int32),
        pltpu.VMEM((CHH, 16), jnp.float32),
        pltpu.VMEM((16, 16), jnp.float32),
        pltpu.VMEM((SA, 16), jnp.float32),
        pltpu.SemaphoreType.DMA,
        pltpu.SemaphoreType.DMA,
    ],
)
def _sc_hist(dst_hbm, deg_hbm,
             hist_sp, didx0, didx1, didx_t, ones, ones_t, hrows, s0, s1):
    cid = lax.axis_index("c")
    sid = lax.axis_index("s")
    zeros16 = jnp.zeros((16,), jnp.float32)
    ones16 = jnp.ones((16,), jnp.float32)
    NCH = EPT_B // CHH

    @pl.loop(0, SA)
    def _(r):
        hrows[r, :] = zeros16

    @pl.loop(0, CHH)
    def _(r):
        ones[r, :] = ones16

    @pl.loop(0, 16)
    def _(r):
        ones_t[r, :] = ones16

    pltpu.sync_copy(hrows, hist_sp.at[pl.ds(sid * SA, SA), :])
    plsc.subcore_barrier()

    ebase = cid * (E // NC) + sid * EPT_B
    pltpu.sync_copy(dst_hbm.at[pl.ds(ebase, CHH)], didx0)

    @pl.loop(0, NCH // 2)
    def _(c):
        pltpu.async_copy(ones, hist_sp.at[didx0], s0, add=True)
        pltpu.sync_copy(dst_hbm.at[pl.ds(ebase + (2 * c + 1) * CHH, CHH)],
                        didx1)
        pltpu.async_copy(ones, hist_sp.at[didx1], s1, add=True)
        pltpu.make_async_copy(ones, hist_sp.at[didx0], s0).wait()

        @pl.when(c < NCH // 2 - 1)
        def _():
            pltpu.sync_copy(dst_hbm.at[pl.ds(ebase + (2 * c + 2) * CHH, CHH)],
                            didx0)

        pltpu.make_async_copy(ones, hist_sp.at[didx1], s1).wait()

    n_tail = EPT_B - NCH * CHH
    pltpu.sync_copy(dst_hbm.at[pl.ds(ebase + EPT_B - n_tail, n_tail)], didx_t)
    pltpu.sync_copy(ones_t, hist_sp.at[didx_t], add=True)
    plsc.subcore_barrier()

    pltpu.sync_copy(hist_sp.at[pl.ds(sid * SA, SA), :], hrows)
    pltpu.sync_copy(hrows, deg_hbm.at[pl.ds(cid * NP + sid * SA, SA), :])


@functools.partial(
    pl.kernel,
    out_type=(
        jax.ShapeDtypeStruct((NP, D), jnp.float32),
        jax.ShapeDtypeStruct((NP, 16), jnp.float32),
    ),
    mesh=_mesh,
    compiler_params=_sc_params,
    scratch_types=[
        pltpu.VMEM((SY, 16), jnp.float32),
        pltpu.VMEM((SY, 16), jnp.float32),
        pltpu.VMEM((SY, D), jnp.float32),
        pltpu.SemaphoreType.DMA,
    ],
)
def _sc_y(x_hbm, deg_hbm, y_hbm, dis_hbm, h0, h1, xbuf, xs):
    cid = lax.axis_index("c")
    sid = lax.axis_index("s")
    wid = sid * NC + cid
    rbase = wid * SY
    last = NS * NC - 1
    SL = N - last * SY

    @pl.when(wid < last)
    def _():
        pltpu.async_copy(x_hbm.at[pl.ds(rbase, SY), :], xbuf, xs)

    @pl.when(wid == last)
    def _():
        pltpu.async_copy(x_hbm.at[pl.ds(rbase, SL), :],
                         xbuf.at[pl.ds(0, SL), :], xs)

    pltpu.sync_copy(deg_hbm.at[pl.ds(rbase, SY), :], h0)
    pltpu.sync_copy(deg_hbm.at[pl.ds(NP + rbase, SY), :], h1)

    @pl.loop(0, SY)
    def _(r):
        disv = _rsqrt16(h0[r, :] + h1[r, :] + 1.0)
        h0[r, :] = disv

    pltpu.sync_copy(h0, dis_hbm.at[pl.ds(rbase, SY), :])
    nrows = jnp.where(wid == last, SL, SY)

    @pl.when(wid < last)
    def _():
        pltpu.make_async_copy(x_hbm.at[pl.ds(rbase, SY), :], xbuf, xs).wait()

    @pl.when(wid == last)
    def _():
        pltpu.make_async_copy(x_hbm.at[pl.ds(rbase, SL), :],
                              xbuf.at[pl.ds(0, SL), :], xs).wait()

    @pl.loop(0, nrows)
    def _(r):
        disv = h0[r, :]
        for k in range(D // 16):
            xbuf[r, pl.ds(k * 16, 16)] = xbuf[r, pl.ds(k * 16, 16)] * disv

    @pl.when(wid < last)
    def _():
        pltpu.sync_copy(xbuf, y_hbm.at[pl.ds(rbase, SY), :])

    @pl.when(wid == last)
    def _():
        pltpu.sync_copy(xbuf.at[pl.ds(0, SL), :],
                        y_hbm.at[pl.ds(rbase, SL), :])


CB = 120
NCB = EPT_B // CB
NTR = NCB // 3
TAIL_B = EPT_B - NCB * CB


@functools.partial(
    pl.kernel,
    out_type=jax.ShapeDtypeStruct((NC * NP, D), jnp.float32),
    mesh=_mesh,
    compiler_params=_sc_params,
    scratch_types=[
        pltpu.MemorySpace.VMEM_SHARED((NP, D), jnp.float32),
        pltpu.VMEM((2, CB), jnp.int32),
        pltpu.VMEM((2, CB), jnp.int32),
        pltpu.VMEM((2, CB), jnp.int32),
        pltpu.VMEM((2, TAIL_B), jnp.int32),
        pltpu.VMEM((CB, D), jnp.float32),
        pltpu.VMEM((CB, D), jnp.float32),
        pltpu.VMEM((CB, D), jnp.float32),
        pltpu.SemaphoreType.DMA,
        pltpu.SemaphoreType.DMA,
        pltpu.SemaphoreType.DMA,
        pltpu.SemaphoreType.DMA,
        pltpu.SemaphoreType.DMA,
        pltpu.SemaphoreType.DMA,
    ],
)
def _sc_agg(y_hbm, eidx_hbm, acc_hbm,
            acc_sp, eidx0, eidx1, eidx2, eidx_t, rows0, rows1, rows2,
            g0, g1, g2, s0, s1, s2):
    cid = lax.axis_index("c")
    sid = lax.axis_index("s")
    zeros16 = jnp.zeros((16,), jnp.float32)
    bufs = [(eidx0, rows0, g0, s0), (eidx1, rows1, g1, s1),
            (eidx2, rows2, g2, s2)]

    @pl.loop(0, CB)
    def _(r):
        for k in range(D // 16):
            rows0[r, pl.ds(k * 16, 16)] = zeros16

    for off in (0, CB, 2 * CB, 3 * CB, 4 * CB, SB - CB):
        pltpu.sync_copy(rows0, acc_sp.at[pl.ds(sid * SB + off, CB), :])
    plsc.subcore_barrier()

    ebase = cid * (E // NC) + sid * EPT_B

    def load_idx(c, eidx):
        pltpu.sync_copy(eidx_hbm.at[:, pl.ds(ebase + c * CB, CB)], eidx)

    for j, (eidx, rows, g, _) in enumerate(bufs):
        load_idx(j, eidx)
        pltpu.async_copy(y_hbm.at[eidx.at[0]], rows, g)

    @pl.loop(0, NTR)
    def _(c):
        for j, (eidx, rows, g, sc) in enumerate(bufs):
            pltpu.make_async_copy(y_hbm.at[eidx.at[0]], rows, g).wait()
            pltpu.async_copy(rows, acc_sp.at[eidx.at[1]], sc, add=True)

        for j, (eidx, rows, g, sc) in enumerate(bufs):
            @pl.when(c < NTR - 1)
            def _(eidx=eidx, rows=rows, g=g, sc=sc, j=j):
                pltpu.make_async_copy(rows, acc_sp.at[eidx.at[1]], sc).wait()
                load_idx(3 * c + 3 + j, eidx)
                pltpu.async_copy(y_hbm.at[eidx.at[0]], rows, g)

    for eidx, rows, g, sc in bufs:
        pltpu.make_async_copy(rows, acc_sp.at[eidx.at[1]], sc).wait()

    for c in range(NTR * 3, NCB):
        load_idx(c, eidx0)
        pltpu.async_copy(y_hbm.at[eidx0.at[0]], rows0, g0).wait()
        pltpu.sync_copy(rows0, acc_sp.at[eidx0.at[1]], add=True)

    b = ebase + NCB * CB
    pltpu.sync_copy(eidx_hbm.at[:, pl.ds(b, TAIL_B)], eidx_t)
    pltpu.async_copy(y_hbm.at[eidx_t.at[0]],
                     rows0.at[pl.ds(0, TAIL_B), :], g0).wait()
    pltpu.sync_copy(rows0.at[pl.ds(0, TAIL_B), :],
                    acc_sp.at[eidx_t.at[1]], add=True)
    plsc.subcore_barrier()

    rbase = sid * SB
    for off in (0, CB, 2 * CB, 3 * CB, 4 * CB, SB - CB):
        pltpu.sync_copy(acc_sp.at[pl.ds(rbase + off, CB), :], rows0)
        pltpu.sync_copy(rows0, acc_hbm.at[pl.ds(cid * NP + rbase + off, CB), :])


def _tc_body(y_ref, dis_ref, acc_ref, batch_ref,
             wg_ref, bg_ref, gm_ref, bt_ref, wf_ref, bf_ref, o_ref):
    y = y_ref[...][:N, :]
    disc = dis_ref[...][:N, 0:1]
    acc = acc_ref[0, :N, :] + acc_ref[1, :N, :]
    agg = disc * (acc + y)
    out = jnp.dot(agg, wg_ref[...], preferred_element_type=jnp.float32)
    out = out + bg_ref[...]
    mean = jnp.mean(out, axis=0, keepdims=True)
    var = jnp.mean(out * out, axis=0, keepdims=True) - mean * mean
    xb = (out - mean) * lax.rsqrt(var + 1e-5) * gm_ref[...] + bt_ref[...]
    xe = jnp.where(xb > 0, xb, jnp.exp(jnp.minimum(xb, 0.0)) - 1.0)
    onehot = (lax.broadcasted_iota(jnp.int32, (G, N), 0)
              == batch_ref[...]).astype(jnp.float32)
    sums = jnp.dot(onehot, xe, preferred_element_type=jnp.float32)
    cnt = jnp.sum(onehot, axis=1, keepdims=True)
    pooled = sums / jnp.maximum(cnt, 1.0)
    o_ref[...] = (jnp.dot(pooled, wf_ref[...],
                          preferred_element_type=jnp.float32) + bf_ref[...])


_tc_final = pl.pallas_call(
    _tc_body,
    out_shape=jax.ShapeDtypeStruct((G, D), jnp.float32),
)


def kernel(x, edge_index, batch, W_gcn, b_gcn, gamma, beta, W_fc, b_fc):
    dst = edge_index[1]
    degp = _sc_hist(dst)
    y, dis = _sc_y(x, degp)
    acc = _sc_agg(y, edge_index)
    return _tc_final(y, dis, acc.reshape(NC, NP, D),
                     batch.reshape(1, N), W_gcn, b_gcn.reshape(1, D),
                     gamma.reshape(1, D), beta.reshape(1, D),
                     W_fc, b_fc.reshape(1, D))


# speedup vs baseline: 1.1008x; 1.0122x over previous
import functools

import jax
import jax.numpy as jnp
from jax import lax
from jax.experimental import pallas as pl
from jax.experimental.pallas import tpu as pltpu, tpu_sc as plsc

N = 10000
E = 320000
D = 128
G = 64
NP = 10240
NS = 16
NC = 2
SA = NP // NS
SY = NP // (NS * NC)
SB = NP // NS
EPT_B = E // (NS * NC)
CH = 128
CHH = 128

_mesh = plsc.VectorSubcoreMesh(core_axis_name="c", subcore_axis_name="s")
_sc_params = pltpu.CompilerParams(use_tc_tiling_on_sc=False)


def _rsqrt16(v):
    i = lax.bitcast_convert_type(v, jnp.int32)
    i = jnp.int32(0x5F3759DF) - lax.shift_right_logical(i, 1)
    z = lax.bitcast_convert_type(i, jnp.float32)
    for _ in range(3):
        z = z * (1.5 - 0.5 * v * z * z)
    return z


@functools.partial(
    pl.kernel,
    out_type=jax.ShapeDtypeStruct((NC * NP, 16), jnp.float32),
    mesh=_mesh,
    compiler_params=_sc_params,
    scratch_types=[
        pltpu.MemorySpace.VMEM_SHARED((NP, 16), jnp.float32),
        pltpu.VMEM((CHH,), jnp.int32),
        pltpu.VMEM((CHH,), jnp.int32),
        pltpu.VMEM((16,), jnp.int32),
        pltpu.VMEM((CHH, 16), jnp.float32),
        pltpu.VMEM((16, 16), jnp.float32),
        pltpu.VMEM((SA, 16), jnp.float32),
        pltpu.SemaphoreType.DMA,
        pltpu.SemaphoreType.DMA,
    ],
)
def _sc_hist(dst_hbm, deg_hbm,
             hist_sp, didx0, didx1, didx_t, ones, ones_t, hrows, s0, s1):
    cid = lax.axis_index("c")
    sid = lax.axis_index("s")
    zeros16 = jnp.zeros((16,), jnp.float32)
    ones16 = jnp.ones((16,), jnp.float32)
    NCH = EPT_B // CHH

    @pl.loop(0, SA)
    def _(r):
        hrows[r, :] = zeros16

    @pl.loop(0, CHH)
    def _(r):
        ones[r, :] = ones16

    @pl.loop(0, 16)
    def _(r):
        ones_t[r, :] = ones16

    pltpu.sync_copy(hrows, hist_sp.at[pl.ds(sid * SA, SA), :])
    plsc.subcore_barrier()

    ebase = cid * (E // NC) + sid * EPT_B
    pltpu.sync_copy(dst_hbm.at[pl.ds(ebase, CHH)], didx0)

    @pl.loop(0, NCH // 2)
    def _(c):
        pltpu.async_copy(ones, hist_sp.at[didx0], s0, add=True)
        pltpu.sync_copy(dst_hbm.at[pl.ds(ebase + (2 * c + 1) * CHH, CHH)],
                        didx1)
        pltpu.async_copy(ones, hist_sp.at[didx1], s1, add=True)
        pltpu.make_async_copy(ones, hist_sp.at[didx0], s0).wait()

        @pl.when(c < NCH // 2 - 1)
        def _():
            pltpu.sync_copy(dst_hbm.at[pl.ds(ebase + (2 * c + 2) * CHH, CHH)],
                            didx0)

        pltpu.make_async_copy(ones, hist_sp.at[didx1], s1).wait()

    n_tail = EPT_B - NCH * CHH
    pltpu.sync_copy(dst_hbm.at[pl.ds(ebase + EPT_B - n_tail, n_tail)], didx_t)
    pltpu.sync_copy(ones_t, hist_sp.at[didx_t], add=True)
    plsc.subcore_barrier()

    pltpu.sync_copy(hist_sp.at[pl.ds(sid * SA, SA), :], hrows)
    pltpu.sync_copy(hrows, deg_hbm.at[pl.ds(cid * NP + sid * SA, SA), :])


@functools.partial(
    pl.kernel,
    out_type=(
        jax.ShapeDtypeStruct((NP, D), jnp.float32),
        jax.ShapeDtypeStruct((NP, 16), jnp.float32),
    ),
    mesh=_mesh,
    compiler_params=_sc_params,
    scratch_types=[
        pltpu.VMEM((SY, 16), jnp.float32),
        pltpu.VMEM((SY, 16), jnp.float32),
        pltpu.VMEM((SY, D), jnp.float32),
        pltpu.SemaphoreType.DMA,
    ],
)
def _sc_y(x_hbm, deg_hbm, y_hbm, dis_hbm, h0, h1, xbuf, xs):
    cid = lax.axis_index("c")
    sid = lax.axis_index("s")
    wid = sid * NC + cid
    rbase = wid * SY
    last = NS * NC - 1
    SL = N - last * SY

    @pl.when(wid < last)
    def _():
        pltpu.async_copy(x_hbm.at[pl.ds(rbase, SY), :], xbuf, xs)

    @pl.when(wid == last)
    def _():
        pltpu.async_copy(x_hbm.at[pl.ds(rbase, SL), :],
                         xbuf.at[pl.ds(0, SL), :], xs)

    pltpu.sync_copy(deg_hbm.at[pl.ds(rbase, SY), :], h0)
    pltpu.sync_copy(deg_hbm.at[pl.ds(NP + rbase, SY), :], h1)

    @pl.loop(0, SY)
    def _(r):
        disv = _rsqrt16(h0[r, :] + h1[r, :] + 1.0)
        h0[r, :] = disv

    pltpu.sync_copy(h0, dis_hbm.at[pl.ds(rbase, SY), :])
    nrows = jnp.where(wid == last, SL, SY)

    @pl.when(wid < last)
    def _():
        pltpu.make_async_copy(x_hbm.at[pl.ds(rbase, SY), :], xbuf, xs).wait()

    @pl.when(wid == last)
    def _():
        pltpu.make_async_copy(x_hbm.at[pl.ds(rbase, SL), :],
                              xbuf.at[pl.ds(0, SL), :], xs).wait()

    @pl.loop(0, nrows)
    def _(r):
        disv = h0[r, :]
        for k in range(D // 16):
            xbuf[r, pl.ds(k * 16, 16)] = xbuf[r, pl.ds(k * 16, 16)] * disv

    @pl.when(wid < last)
    def _():
        pltpu.sync_copy(xbuf, y_hbm.at[pl.ds(rbase, SY), :])

    @pl.when(wid == last)
    def _():
        pltpu.sync_copy(xbuf.at[pl.ds(0, SL), :],
                        y_hbm.at[pl.ds(rbase, SL), :])


CB = 88
NCB = EPT_B // CB
NB4 = 4
NTR = NCB // NB4
TAIL_B = EPT_B - NCB * CB


@functools.partial(
    pl.kernel,
    out_type=jax.ShapeDtypeStruct((NC * NP, D), jnp.float32),
    mesh=_mesh,
    compiler_params=_sc_params,
    scratch_types=[
        pltpu.MemorySpace.VMEM_SHARED((NP, D), jnp.float32),
        pltpu.VMEM((2, CB), jnp.int32),
        pltpu.VMEM((2, CB), jnp.int32),
        pltpu.VMEM((2, CB), jnp.int32),
        pltpu.VMEM((2, CB), jnp.int32),
        pltpu.VMEM((2, TAIL_B), jnp.int32),
        pltpu.VMEM((CB, D), jnp.float32),
        pltpu.VMEM((CB, D), jnp.float32),
        pltpu.VMEM((CB, D), jnp.float32),
        pltpu.VMEM((CB, D), jnp.float32),
        pltpu.SemaphoreType.DMA,
        pltpu.SemaphoreType.DMA,
        pltpu.SemaphoreType.DMA,
        pltpu.SemaphoreType.DMA,
        pltpu.SemaphoreType.DMA,
        pltpu.SemaphoreType.DMA,
        pltpu.SemaphoreType.DMA,
        pltpu.SemaphoreType.DMA,
    ],
)
def _sc_agg(y_hbm, eidx_hbm, acc_hbm,
            acc_sp, eidx0, eidx1, eidx2, eidx3, eidx_t,
            rows0, rows1, rows2, rows3, g0, g1, g2, g3, s0, s1, s2, s3):
    cid = lax.axis_index("c")
    sid = lax.axis_index("s")
    zeros16 = jnp.zeros((16,), jnp.float32)
    bufs = [(eidx0, rows0, g0, s0), (eidx1, rows1, g1, s1),
            (eidx2, rows2, g2, s2), (eidx3, rows3, g3, s3)]

    @pl.loop(0, CB)
    def _(r):
        for k in range(D // 16):
            rows0[r, pl.ds(k * 16, 16)] = zeros16

    nz = (SB + CB - 1) // CB
    for j in range(nz):
        off = min(j * CB, SB - CB)
        pltpu.sync_copy(rows0, acc_sp.at[pl.ds(sid * SB + off, CB), :])
    plsc.subcore_barrier()

    ebase = cid * (E // NC) + sid * EPT_B

    def load_idx(c, eidx):
        pltpu.sync_copy(eidx_hbm.at[:, pl.ds(ebase + c * CB, CB)], eidx)

    for j, (eidx, rows, g, _) in enumerate(bufs):
        load_idx(j, eidx)
        pltpu.async_copy(y_hbm.at[eidx.at[0]], rows, g)

    @pl.loop(0, NTR)
    def _(c):
        for j, (eidx, rows, g, sc) in enumerate(bufs):
            pltpu.make_async_copy(y_hbm.at[eidx.at[0]], rows, g).wait()
            pltpu.async_copy(rows, acc_sp.at[eidx.at[1]], sc, add=True)

        for j, (eidx, rows, g, sc) in enumerate(bufs):
            @pl.when(c < NTR - 1)
            def _(eidx=eidx, rows=rows, g=g, sc=sc, j=j):
                pltpu.make_async_copy(rows, acc_sp.at[eidx.at[1]], sc).wait()
                load_idx(NB4 * c + NB4 + j, eidx)
                pltpu.async_copy(y_hbm.at[eidx.at[0]], rows, g)

    for eidx, rows, g, sc in bufs:
        pltpu.make_async_copy(rows, acc_sp.at[eidx.at[1]], sc).wait()

    for c in range(NTR * NB4, NCB):
        load_idx(c, eidx0)
        pltpu.async_copy(y_hbm.at[eidx0.at[0]], rows0, g0).wait()
        pltpu.sync_copy(rows0, acc_sp.at[eidx0.at[1]], add=True)

    b = ebase + NCB * CB
    pltpu.sync_copy(eidx_hbm.at[:, pl.ds(b, TAIL_B)], eidx_t)
    pltpu.async_copy(y_hbm.at[eidx_t.at[0]],
                     rows0.at[pl.ds(0, TAIL_B), :], g0).wait()
    pltpu.sync_copy(rows0.at[pl.ds(0, TAIL_B), :],
                    acc_sp.at[eidx_t.at[1]], add=True)
    plsc.subcore_barrier()

    rbase = sid * SB
    for j in range((SB + CB - 1) // CB):
        off = min(j * CB, SB - CB)
        pltpu.sync_copy(acc_sp.at[pl.ds(rbase + off, CB), :], rows0)
        pltpu.sync_copy(rows0, acc_hbm.at[pl.ds(cid * NP + rbase + off, CB), :])


def _tc_body(y_ref, dis_ref, acc_ref, batch_ref,
             wg_ref, bg_ref, gm_ref, bt_ref, wf_ref, bf_ref, o_ref):
    y = y_ref[...][:N, :]
    disc = dis_ref[...][:N, 0:1]
    acc = acc_ref[0, :N, :] + acc_ref[1, :N, :]
    agg = disc * (acc + y)
    out = jnp.dot(agg, wg_ref[...], preferred_element_type=jnp.float32)
    out = out + bg_ref[...]
    mean = jnp.mean(out, axis=0, keepdims=True)
    var = jnp.mean(out * out, axis=0, keepdims=True) - mean * mean
    xb = (out - mean) * lax.rsqrt(var + 1e-5) * gm_ref[...] + bt_ref[...]
    xe = jnp.where(xb > 0, xb, jnp.exp(jnp.minimum(xb, 0.0)) - 1.0)
    onehot = (lax.broadcasted_iota(jnp.int32, (G, N), 0)
              == batch_ref[...]).astype(jnp.float32)
    sums = jnp.dot(onehot, xe, preferred_element_type=jnp.float32)
    cnt = jnp.sum(onehot, axis=1, keepdims=True)
    pooled = sums / jnp.maximum(cnt, 1.0)
    o_ref[...] = (jnp.dot(pooled, wf_ref[...],
                          preferred_element_type=jnp.float32) + bf_ref[...])


_tc_final = pl.pallas_call(
    _tc_body,
    out_shape=jax.ShapeDtypeStruct((G, D), jnp.float32),
)


def kernel(x, edge_index, batch, W_gcn, b_gcn, gamma, beta, W_fc, b_fc):
    dst = edge_index[1]
    degp = _sc_hist(dst)
    y, dis = _sc_y(x, degp)
    acc = _sc_agg(y, edge_index)
    return _tc_final(y, dis, acc.reshape(NC, NP, D),
                     batch.reshape(1, N), W_gcn, b_gcn.reshape(1, D),
                     gamma.reshape(1, D), beta.reshape(1, D),
                     W_fc, b_fc.reshape(1, D))
